# baseline (speedup 1.0000x reference)
import functools

import jax
import jax.numpy as jnp
from jax import lax
from jax.experimental import pallas as pl
from jax.experimental.pallas import tpu as pltpu
from jax.experimental.pallas import tpu_sc as plsc


def _mlp_kernel(x_ref, W1_ref, b1_ref, W2_ref, b2_ref,
                Ws1_ref, bs1_ref, Ws2_ref, bs2_ref, x1_ref, xs_ref):
    x = x_ref[...]
    h1 = jax.nn.relu(jnp.dot(x, W1_ref[...], preferred_element_type=jnp.float32)
                     + b1_ref[...])
    x1_ref[...] = jax.nn.relu(
        jnp.dot(h1, W2_ref[...], preferred_element_type=jnp.float32) + b2_ref[...])
    hs = jax.nn.relu(jnp.dot(x, Ws1_ref[...], preferred_element_type=jnp.float32)
                     + bs1_ref[...])
    xs_ref[...] = jax.nn.relu(
        jnp.dot(hs, Ws2_ref[...], preferred_element_type=jnp.float32) + bs2_ref[...])


_R = 4
_JC = 128


def _sc_msg_kernel(A_hbm, W_hbm, x1_hbm, xs_hbm, out_hbm,
                   x1v, wv0, wv1, av, xsv, outv, sem0, sem1,
                   *, n, f, n_workers):
    rows_per_worker = (2 * n) // n_workers
    c = lax.axis_index("c")
    s = lax.axis_index("s")
    wid = c * 16 + s
    per_batch = n_workers // 2
    batch = wid // per_batch
    i0 = (wid % per_batch) * rows_per_worker
    n_chunks = n // _JC

    pltpu.sync_copy(x1_hbm.at[batch], x1v)

    def w_copy(ib, jc, wv, sem):
        class _Multi:
            def __init__(self, cps):
                self.cps = cps

            def start(self):
                for cp in self.cps:
                    cp.start()

            def wait(self):
                for cp in self.cps:
                    cp.wait()
        return _Multi([
            pltpu.make_async_copy(
                W_hbm.at[batch, ib + r, pl.ds(jc * _JC, _JC), :],
                wv.at[pl.ds(r * _JC, _JC)], sem)
            for r in range(_R)
        ])

    def compute(jc, wv, carry):
        accs, asums = list(carry[0]), list(carry[1])
        for r0 in range(0, _R, 2):
            acc0, acc1 = accs[r0], accs[r0 + 1]
            as0, as1 = asums[r0], asums[r0 + 1]
            for jb in range(_JC // 16):
                jj = jc * _JC + jb * 16
                a0 = av[r0, pl.ds(jj, 16)]
                a1 = av[r0 + 1, pl.ds(jj, 16)]
                as0 = as0 + a0
                as1 = as1 + a1
                for l in range(16):
                    x116 = x1v[pl.ds((jj + l) * f, f)]
                    w0 = wv[r0 * _JC + jb * 16 + l, :]
                    w1 = wv[(r0 + 1) * _JC + jb * 16 + l, :]
                    acc0 = acc0 + (w0 * x116) * a0[l]
                    acc1 = acc1 + (w1 * x116) * a1[l]
            accs[r0], accs[r0 + 1] = acc0, acc1
            asums[r0], asums[r0 + 1] = as0, as1
        return (tuple(accs), tuple(asums))

    def group(g, _):
        ib = i0 + g * _R
        pltpu.sync_copy(A_hbm.at[batch, pl.ds(ib, _R), :], av)
        pltpu.sync_copy(xs_hbm.at[batch, pl.ds(ib, _R), :], xsv)

        zero = tuple(jnp.zeros((f,), jnp.float32) for _ in range(_R))
        w_copy(ib, 0, wv0, sem0).start()

        def super_chunk(jcc, carry):
            jc0 = jcc * 2
            w_copy(ib, jc0 + 1, wv1, sem1).start()
            w_copy(ib, jc0, wv0, sem0).wait()
            carry = compute(jc0, wv0, carry)

            @pl.when(jcc + 1 < n_chunks // 2)
            def _():
                w_copy(ib, jc0 + 2, wv0, sem0).start()
            w_copy(ib, jc0 + 1, wv1, sem1).wait()
            return compute(jc0 + 1, wv1, carry)

        accs, asums = lax.fori_loop(0, n_chunks // 2, super_chunk, (zero, zero))

        ones = jnp.ones((f,), jnp.float32)
        iota = lax.iota(jnp.int32, f)
        for r in range(_R):
            sv = asums[r]
            for k in (8, 4, 2, 1):
                sv = sv + jnp.take(sv, iota ^ k)
            scale16 = jnp.where(sv != 0.0, ones / sv, 0.0)
            outv[r, :] = accs[r] * scale16 + xsv[r, :]
        pltpu.sync_copy(outv, out_hbm.at[batch, pl.ds(ib, _R), :])
        return 0

    lax.fori_loop(0, rows_per_worker // _R, group, 0)


@jax.jit
def kernel(A, W, x, W1, b1, W2, b2, Ws1, bs1, Ws2, bs2):
    b, n, din = x.shape
    f = W.shape[-1]

    x2d = x.reshape(b * n, din)
    x1f, xsf = pl.pallas_call(
        _mlp_kernel,
        out_shape=(
            jax.ShapeDtypeStruct((b * n, f), jnp.float32),
            jax.ShapeDtypeStruct((b * n, f), jnp.float32),
        ),
    )(x2d, W1, b1.reshape(1, f), W2, b2.reshape(1, f),
      Ws1, bs1.reshape(1, f), Ws2, bs2.reshape(1, f))
    x1 = x1f.reshape(b, n * f)
    xs = xsf.reshape(b, n, f)

    n_workers = 32
    mesh = plsc.VectorSubcoreMesh(core_axis_name="c", subcore_axis_name="s")
    sc = functools.partial(
        pl.kernel,
        mesh=mesh,
        compiler_params=pltpu.CompilerParams(use_tc_tiling_on_sc=False),
        out_type=jax.ShapeDtypeStruct((b, n, f), jnp.float32),
        scratch_types=[
            pltpu.VMEM((n * f,), jnp.float32),
            pltpu.VMEM((_R * _JC, f), jnp.float32),
            pltpu.VMEM((_R * _JC, f), jnp.float32),
            pltpu.VMEM((_R, n), jnp.float32),
            pltpu.VMEM((_R, f), jnp.float32),
            pltpu.VMEM((_R, f), jnp.float32),
            pltpu.SemaphoreType.DMA,
            pltpu.SemaphoreType.DMA,
        ],
    )(functools.partial(_sc_msg_kernel, n=n, f=f, n_workers=n_workers))
    x2 = sc(A, W, x1, xs)

    return (W, x2)


# speedup vs baseline: 3.4841x; 3.4841x over previous
import functools

import jax
import jax.numpy as jnp
from jax.experimental import pallas as pl


def _mlp_kernel(x_ref, W1_ref, b1_ref, W2_ref, b2_ref,
                Ws1_ref, bs1_ref, Ws2_ref, bs2_ref, x1_ref, xs_ref):
    x = x_ref[...]
    h1 = jax.nn.relu(jnp.dot(x, W1_ref[...], preferred_element_type=jnp.float32)
                     + b1_ref[...])
    x1_ref[...] = jax.nn.relu(
        jnp.dot(h1, W2_ref[...], preferred_element_type=jnp.float32) + b2_ref[...])
    hs = jax.nn.relu(jnp.dot(x, Ws1_ref[...], preferred_element_type=jnp.float32)
                     + bs1_ref[...])
    xs_ref[...] = jax.nn.relu(
        jnp.dot(hs, Ws2_ref[...], preferred_element_type=jnp.float32) + bs2_ref[...])


def _msg_kernel(A_ref, W_ref, x1_ref, xs_ref, E_ref, out_ref, *, f, ch):
    a = A_ref[0]
    nf = W_ref.shape[2]
    asum = jnp.sum(a, axis=1, keepdims=True)
    scale = jnp.where(asum != 0.0, 1.0 / asum, 0.0)
    emat = E_ref[...]
    acc = jnp.zeros((a.shape[0], 128), jnp.float32)
    for c in range(nf // ch):
        w = W_ref[0, :, c * ch:(c + 1) * ch]
        xv = x1_ref[0, :, c * ch:(c + 1) * ch]
        ac = a[:, c * (ch // f):(c + 1) * (ch // f)]
        ar = jnp.dot(ac, emat, preferred_element_type=jnp.float32)
        t = w * xv * ar
        width = ch
        while width > 128:
            half = width // 2
            t = t[:, :half] + t[:, half:width]
            width = half
        acc = acc + t
    r = acc[:, 0:f]
    for g in range(1, 128 // f):
        r = r + acc[:, g * f:(g + 1) * f]
    out_ref[0] = r * scale + xs_ref[0]


@jax.jit
def kernel(A, W, x, W1, b1, W2, b2, Ws1, bs1, Ws2, bs2):
    b, n, din = x.shape
    f = W.shape[-1]

    x2d = x.reshape(b * n, din)
    x1f, xsf = pl.pallas_call(
        _mlp_kernel,
        out_shape=(
            jax.ShapeDtypeStruct((b * n, f), jnp.float32),
            jax.ShapeDtypeStruct((b * n, f), jnp.float32),
        ),
    )(x2d, W1, b1.reshape(1, f), W2, b2.reshape(1, f),
      Ws1, bs1.reshape(1, f), Ws2, bs2.reshape(1, f))
    x1 = x1f.reshape(b, 1, n * f)
    xs = xsf.reshape(b, n, f)

    ch = 512
    E = (jnp.arange(ch, dtype=jnp.int32)[None, :] // f
         == jnp.arange(ch // f, dtype=jnp.int32)[:, None]).astype(jnp.float32)

    Wf = W.reshape(b, n, n * f)
    BI = 128
    x2 = pl.pallas_call(
        functools.partial(_msg_kernel, f=f, ch=ch),
        grid=(b, n // BI),
        in_specs=[
            pl.BlockSpec((1, BI, n), lambda bb, ii: (bb, ii, 0)),
            pl.BlockSpec((1, BI, n * f), lambda bb, ii: (bb, ii, 0)),
            pl.BlockSpec((1, 1, n * f), lambda bb, ii: (bb, 0, 0)),
            pl.BlockSpec((1, BI, f), lambda bb, ii: (bb, ii, 0)),
            pl.BlockSpec((ch // f, ch), lambda bb, ii: (0, 0)),
        ],
        out_specs=pl.BlockSpec((1, BI, f), lambda bb, ii: (bb, ii, 0)),
        out_shape=jax.ShapeDtypeStruct((b, n, f), jnp.float32),
    )(A, Wf, x1, xs, E)

    return (W, x2)
